# baseline (speedup 1.0000x reference)
import functools

import jax
import jax.numpy as jnp
from jax import lax
from jax.experimental import pallas as pl
from jax.experimental.pallas import tpu as pltpu
from jax.experimental.pallas import tpu_sc as plsc

N_NODES = 10000
D = 128
N_EDGES = 320000

NW = 32
CHUNK = 128
NCHUNK = 80
PER_W = CHUNK * NCHUNK
E_PAD = PER_W * NW

ROW_BLOCK = 1000


def _mlp_body(x_ref, w1_ref, b1_ref, w2_ref, b2_ref, h_ref):
    h1 = jnp.dot(x_ref[...], w1_ref[...], preferred_element_type=jnp.float32)
    h1 = jnp.maximum(h1 + b1_ref[...], 0.0)
    h2 = jnp.dot(h1, w2_ref[...], preferred_element_type=jnp.float32)
    h_ref[...] = jnp.maximum(h2 + b2_ref[...], 0.0)


def _mlp(x, W1, b1, W2, b2):
    grid = (N_NODES // ROW_BLOCK,)
    return pl.pallas_call(
        _mlp_body,
        grid=grid,
        in_specs=[
            pl.BlockSpec((ROW_BLOCK, D), lambda i: (i, 0)),
            pl.BlockSpec((D, D), lambda i: (0, 0)),
            pl.BlockSpec((1, D), lambda i: (0, 0)),
            pl.BlockSpec((D, D), lambda i: (0, 0)),
            pl.BlockSpec((1, D), lambda i: (0, 0)),
        ],
        out_specs=pl.BlockSpec((ROW_BLOCK, D), lambda i: (i, 0)),
        out_shape=jax.ShapeDtypeStruct((N_NODES, D), jnp.float32),
    )(x, W1, b1.reshape(1, D), W2, b2.reshape(1, D))


def _edge_dot_body(h_hbm, src_hbm, dst_hbm, pred_hbm,
                   sidx, didx, srows0, srows1, drows0, drows1,
                   accbuf, outb0, outb1,
                   sem_s0, sem_s1, sem_d0, sem_d1, sem_o0, sem_o1):
    wid = lax.axis_index("s") * 2 + lax.axis_index("c")
    base = wid * PER_W
    lanes = lax.iota(jnp.int32, 16)

    srows = (srows0, srows1)
    drows = (drows0, drows1)
    outb = (outb0, outb1)
    sem_s = (sem_s0, sem_s1)
    sem_d = (sem_d0, sem_d1)
    sem_o = (sem_o0, sem_o1)

    pltpu.sync_copy(src_hbm.at[pl.ds(wid * NCHUNK, NCHUNK)], sidx)
    pltpu.sync_copy(dst_hbm.at[pl.ds(wid * NCHUNK, NCHUNK)], didx)

    pltpu.async_copy(h_hbm.at[sidx.at[0]], srows0, sem_s0)
    pltpu.async_copy(h_hbm.at[didx.at[0]], drows0, sem_d0)

    def compute_chunk(sr, dr, ob):
        def group_body(g, gcarry):
            ebase = g * 16
            for r in range(16):
                e = ebase + r
                acc = sr[e, pl.ds(0, 16)] * dr[e, pl.ds(0, 16)]
                for j in range(1, D // 16):
                    acc = acc + sr[e, pl.ds(j * 16, 16)] * dr[e, pl.ds(j * 16, 16)]
                accbuf[pl.ds(r * 16, 16)] = acc
            rowbase = lanes * 16
            out16 = plsc.load_gather(accbuf, [rowbase])
            for l in range(1, 16):
                out16 = out16 + plsc.load_gather(accbuf, [rowbase + l])
            ob[pl.ds(ebase, 16)] = out16
            return gcarry

        lax.fori_loop(0, CHUNK // 16, group_body, 0)

    def pair_body(c2, carry):
        for b in range(2):
            c = c2 * 2 + b
            nb = 1 - b

            @pl.when(c + 1 < NCHUNK)
            def _():
                pltpu.async_copy(h_hbm.at[sidx.at[c + 1]], srows[nb], sem_s[nb])
                pltpu.async_copy(h_hbm.at[didx.at[c + 1]], drows[nb], sem_d[nb])

            pltpu.make_async_copy(h_hbm.at[sidx.at[c]], srows[b], sem_s[b]).wait()
            pltpu.make_async_copy(h_hbm.at[didx.at[c]], drows[b], sem_d[b]).wait()

            @pl.when(c >= 2)
            def _():
                pltpu.make_async_copy(
                    outb[b], pred_hbm.at[pl.ds(base, CHUNK)], sem_o[b]).wait()

            compute_chunk(srows[b], drows[b], outb[b])
            pltpu.async_copy(
                outb[b], pred_hbm.at[pl.ds(base + c * CHUNK, CHUNK)], sem_o[b])
        return carry

    lax.fori_loop(0, NCHUNK // 2, pair_body, 0)

    for b in range(2):
        pltpu.make_async_copy(
            outb[b], pred_hbm.at[pl.ds(base, CHUNK)], sem_o[b]).wait()


def _edge_dots(h, src2d, dst2d):
    mesh = plsc.VectorSubcoreMesh(core_axis_name="c", subcore_axis_name="s")
    k = functools.partial(
        pl.kernel,
        out_type=jax.ShapeDtypeStruct((E_PAD,), jnp.float32),
        mesh=mesh,
        scratch_types=[
            pltpu.VMEM((NCHUNK, CHUNK), jnp.int32),
            pltpu.VMEM((NCHUNK, CHUNK), jnp.int32),
            pltpu.VMEM((CHUNK, D), jnp.float32),
            pltpu.VMEM((CHUNK, D), jnp.float32),
            pltpu.VMEM((CHUNK, D), jnp.float32),
            pltpu.VMEM((CHUNK, D), jnp.float32),
            pltpu.VMEM((256,), jnp.float32),
            pltpu.VMEM((CHUNK,), jnp.float32),
            pltpu.VMEM((CHUNK,), jnp.float32),
            pltpu.SemaphoreType.DMA,
            pltpu.SemaphoreType.DMA,
            pltpu.SemaphoreType.DMA,
            pltpu.SemaphoreType.DMA,
            pltpu.SemaphoreType.DMA,
            pltpu.SemaphoreType.DMA,
        ],
        compiler_params=pltpu.CompilerParams(needs_layout_passes=False),
    )(_edge_dot_body)
    return k(h, src2d, dst2d)


def kernel(x, edge_index, edge_label, W1, b1, W2, b2):
    h = _mlp(x, W1, b1, W2, b2)
    ei = edge_index.astype(jnp.int32)
    pad = E_PAD - N_EDGES
    src2d = jnp.pad(ei[0], (0, pad)).reshape(NW * NCHUNK, CHUNK)
    dst2d = jnp.pad(ei[1], (0, pad)).reshape(NW * NCHUNK, CHUNK)
    pred_pad = _edge_dots(h, src2d, dst2d)
    return (pred_pad[:N_EDGES], edge_label)


# speedup vs baseline: 1.1892x; 1.1892x over previous
import functools

import jax
import jax.numpy as jnp
from jax import lax
from jax.experimental import pallas as pl
from jax.experimental.pallas import tpu as pltpu
from jax.experimental.pallas import tpu_sc as plsc

N_NODES = 10000
D = 128
DW = D // 2
N_EDGES = 320000

NW = 32
CHUNK = 128
NCHUNK = 80
PER_W = CHUNK * NCHUNK
E_PAD = PER_W * NW

ROW_BLOCK = 1000


def _mlp_body(x_ref, w1_ref, b1_ref, w2_ref, b2_ref, h_ref):
    h1 = jnp.dot(x_ref[...], w1_ref[...], preferred_element_type=jnp.float32)
    h1 = jnp.maximum(h1 + b1_ref[...], 0.0)
    h2 = jnp.dot(h1, w2_ref[...], preferred_element_type=jnp.float32)
    h_ref[...] = jnp.maximum(h2 + b2_ref[...], 0.0).astype(jnp.bfloat16)


def _mlp(x, W1, b1, W2, b2):
    grid = (N_NODES // ROW_BLOCK,)
    return pl.pallas_call(
        _mlp_body,
        grid=grid,
        in_specs=[
            pl.BlockSpec((ROW_BLOCK, D), lambda i: (i, 0)),
            pl.BlockSpec((D, D), lambda i: (0, 0)),
            pl.BlockSpec((1, D), lambda i: (0, 0)),
            pl.BlockSpec((D, D), lambda i: (0, 0)),
            pl.BlockSpec((1, D), lambda i: (0, 0)),
        ],
        out_specs=pl.BlockSpec((ROW_BLOCK, D), lambda i: (i, 0)),
        out_shape=jax.ShapeDtypeStruct((N_NODES, D), jnp.bfloat16),
    )(x, W1, b1.reshape(1, D), W2, b2.reshape(1, D))


def _edge_dot_body(h_hbm, src_hbm, dst_hbm, pred_hbm,
                   sidx, didx, srows, drows, accbuf, outbuf, sem1, sem2):
    wid = lax.axis_index("s") * 2 + lax.axis_index("c")
    base = wid * PER_W
    lanes = lax.iota(jnp.int32, 16)

    pltpu.sync_copy(src_hbm.at[pl.ds(wid * NCHUNK, NCHUNK)], sidx)
    pltpu.sync_copy(dst_hbm.at[pl.ds(wid * NCHUNK, NCHUNK)], didx)

    def chunk_body(c, carry):
        cp1 = pltpu.async_copy(h_hbm.at[sidx.at[c]], srows, sem1)
        cp2 = pltpu.async_copy(h_hbm.at[didx.at[c]], drows, sem2)
        cp1.wait()
        cp2.wait()

        def group_body(g, gcarry):
            ebase = g * 16
            for r in range(16):
                e = ebase + r
                acc = None
                for j in range(DW // 16):
                    a32 = plsc.bitcast(srows[e, pl.ds(j * 16, 16)], jnp.bfloat16)
                    b32 = plsc.bitcast(drows[e, pl.ds(j * 16, 16)], jnp.bfloat16)
                    a0, a1 = plsc.unpack(a32, format=plsc.PackFormat.INTERLEAVED)
                    b0, b1 = plsc.unpack(b32, format=plsc.PackFormat.INTERLEAVED)
                    term = a0 * b0 + a1 * b1
                    acc = term if acc is None else acc + term
                accbuf[pl.ds(r * 16, 16)] = acc
            rowbase = lanes * 16
            out16 = plsc.load_gather(accbuf, [rowbase])
            for l in range(1, 16):
                out16 = out16 + plsc.load_gather(accbuf, [rowbase + l])
            outbuf[pl.ds(ebase, 16)] = out16
            return gcarry

        lax.fori_loop(0, CHUNK // 16, group_body, 0)
        pltpu.sync_copy(outbuf, pred_hbm.at[pl.ds(base + c * CHUNK, CHUNK)])
        return carry

    lax.fori_loop(0, NCHUNK, chunk_body, 0)


def _edge_dots(h32, src2d, dst2d):
    mesh = plsc.VectorSubcoreMesh(core_axis_name="c", subcore_axis_name="s")
    k = functools.partial(
        pl.kernel,
        out_type=jax.ShapeDtypeStruct((E_PAD,), jnp.float32),
        mesh=mesh,
        scratch_types=[
            pltpu.VMEM((NCHUNK, CHUNK), jnp.int32),
            pltpu.VMEM((NCHUNK, CHUNK), jnp.int32),
            pltpu.VMEM((CHUNK, DW), jnp.int32),
            pltpu.VMEM((CHUNK, DW), jnp.int32),
            pltpu.VMEM((256,), jnp.float32),
            pltpu.VMEM((CHUNK,), jnp.float32),
            pltpu.SemaphoreType.DMA,
            pltpu.SemaphoreType.DMA,
        ],
        compiler_params=pltpu.CompilerParams(
            needs_layout_passes=False, use_tc_tiling_on_sc=False),
    )(_edge_dot_body)
    return k(h32, src2d, dst2d)


def kernel(x, edge_index, edge_label, W1, b1, W2, b2):
    h = _mlp(x, W1, b1, W2, b2)
    h32 = lax.bitcast_convert_type(h.reshape(N_NODES, DW, 2), jnp.int32)
    ei = edge_index.astype(jnp.int32)
    pad = E_PAD - N_EDGES
    src2d = jnp.pad(ei[0], (0, pad)).reshape(NW * NCHUNK, CHUNK)
    dst2d = jnp.pad(ei[1], (0, pad)).reshape(NW * NCHUNK, CHUNK)
    pred_pad = _edge_dots(h32, src2d, dst2d)
    return (pred_pad[:N_EDGES], edge_label)


# speedup vs baseline: 1.2296x; 1.0339x over previous
import functools

import jax
import jax.numpy as jnp
from jax import lax
from jax.experimental import pallas as pl
from jax.experimental.pallas import tpu as pltpu
from jax.experimental.pallas import tpu_sc as plsc

N_NODES = 10000
D = 128
DW = D // 2
N_EDGES = 320000

NW = 32
SUB = 128
GPC = 4
CHUNK = SUB * GPC
NCHUNK = 20
NSUB = NCHUNK * GPC
PER_W = CHUNK * NCHUNK
E_PAD = PER_W * NW

ROW_BLOCK = 1000


def _mlp_body(x_ref, w1_ref, b1_ref, w2_ref, b2_ref, h_ref):
    h1 = jnp.dot(x_ref[...], w1_ref[...], preferred_element_type=jnp.float32)
    h1 = jnp.maximum(h1 + b1_ref[...], 0.0)
    h2 = jnp.dot(h1, w2_ref[...], preferred_element_type=jnp.float32)
    h_ref[...] = jnp.maximum(h2 + b2_ref[...], 0.0).astype(jnp.bfloat16)


def _mlp(x, W1, b1, W2, b2):
    grid = (N_NODES // ROW_BLOCK,)
    return pl.pallas_call(
        _mlp_body,
        grid=grid,
        in_specs=[
            pl.BlockSpec((ROW_BLOCK, D), lambda i: (i, 0)),
            pl.BlockSpec((D, D), lambda i: (0, 0)),
            pl.BlockSpec((1, D), lambda i: (0, 0)),
            pl.BlockSpec((D, D), lambda i: (0, 0)),
            pl.BlockSpec((1, D), lambda i: (0, 0)),
        ],
        out_specs=pl.BlockSpec((ROW_BLOCK, D), lambda i: (i, 0)),
        out_shape=jax.ShapeDtypeStruct((N_NODES, D), jnp.bfloat16),
    )(x, W1, b1.reshape(1, D), W2, b2.reshape(1, D))


def _edge_dot_body(h_hbm, src_hbm, dst_hbm, pred_hbm,
                   sidx, didx, srows, drows, accbuf, outbuf, sem1, sem2):
    wid = lax.axis_index("s") * 2 + lax.axis_index("c")
    base = wid * PER_W
    lanes = lax.iota(jnp.int32, 16)

    pltpu.sync_copy(src_hbm.at[pl.ds(wid * NSUB, NSUB)], sidx)
    pltpu.sync_copy(dst_hbm.at[pl.ds(wid * NSUB, NSUB)], didx)

    def chunk_body(c, carry):
        cps = []
        for g in range(GPC):
            cps.append(pltpu.async_copy(
                h_hbm.at[sidx.at[c * GPC + g]],
                srows.at[pl.ds(g * SUB, SUB)], sem1))
            cps.append(pltpu.async_copy(
                h_hbm.at[didx.at[c * GPC + g]],
                drows.at[pl.ds(g * SUB, SUB)], sem2))
        for cp in cps:
            cp.wait()

        def group_body(g, gcarry):
            ebase = g * 16
            for r in range(16):
                e = ebase + r
                acc = None
                for j in range(DW // 16):
                    a32 = plsc.bitcast(srows[e, pl.ds(j * 16, 16)], jnp.bfloat16)
                    b32 = plsc.bitcast(drows[e, pl.ds(j * 16, 16)], jnp.bfloat16)
                    a0, a1 = plsc.unpack(a32, format=plsc.PackFormat.INTERLEAVED)
                    b0, b1 = plsc.unpack(b32, format=plsc.PackFormat.INTERLEAVED)
                    term = a0 * b0 + a1 * b1
                    acc = term if acc is None else acc + term
                accbuf[pl.ds(r * 16, 16)] = acc
            rowbase = lanes * 16
            out16 = plsc.load_gather(accbuf, [rowbase])
            for l in range(1, 16):
                out16 = out16 + plsc.load_gather(accbuf, [rowbase + l])
            outbuf[pl.ds(ebase, 16)] = out16
            return gcarry

        lax.fori_loop(0, CHUNK // 16, group_body, 0)
        pltpu.sync_copy(outbuf, pred_hbm.at[pl.ds(base + c * CHUNK, CHUNK)])
        return carry

    lax.fori_loop(0, NCHUNK, chunk_body, 0)


def _edge_dots(h32, src2d, dst2d):
    mesh = plsc.VectorSubcoreMesh(core_axis_name="c", subcore_axis_name="s")
    k = functools.partial(
        pl.kernel,
        out_type=jax.ShapeDtypeStruct((E_PAD,), jnp.float32),
        mesh=mesh,
        scratch_types=[
            pltpu.VMEM((NSUB, SUB), jnp.int32),
            pltpu.VMEM((NSUB, SUB), jnp.int32),
            pltpu.VMEM((CHUNK, DW), jnp.int32),
            pltpu.VMEM((CHUNK, DW), jnp.int32),
            pltpu.VMEM((256,), jnp.float32),
            pltpu.VMEM((CHUNK,), jnp.float32),
            pltpu.SemaphoreType.DMA,
            pltpu.SemaphoreType.DMA,
        ],
        compiler_params=pltpu.CompilerParams(
            needs_layout_passes=False, use_tc_tiling_on_sc=False),
    )(_edge_dot_body)
    return k(h32, src2d, dst2d)


def kernel(x, edge_index, edge_label, W1, b1, W2, b2):
    h = _mlp(x, W1, b1, W2, b2)
    h32 = lax.bitcast_convert_type(h.reshape(N_NODES, DW, 2), jnp.int32)
    ei = edge_index.astype(jnp.int32)
    pad = E_PAD - N_EDGES
    src2d = jnp.pad(ei[0], (0, pad)).reshape(NW * NSUB, SUB)
    dst2d = jnp.pad(ei[1], (0, pad)).reshape(NW * NSUB, SUB)
    pred_pad = _edge_dots(h32, src2d, dst2d)
    return (pred_pad[:N_EDGES], edge_label)


# speedup vs baseline: 2.3577x; 1.9175x over previous
import functools

import jax
import jax.numpy as jnp
from jax import lax
from jax.experimental import pallas as pl
from jax.experimental.pallas import tpu as pltpu
from jax.experimental.pallas import tpu_sc as plsc

N_NODES = 10000
D = 128
N_EDGES = 320000

NSC = 2
NTILE = 16
WPT = D // (2 * NTILE) // 2

WORDS = D // NTILE

CHUNK = 4096
ROWS = CHUNK // 128
PER_SC = 163840
NCHUNK = PER_SC // CHUNK
E_PAD = PER_SC * NSC

ROW_BLOCK = 1000


def _mlp_body(x_ref, w1_ref, b1_ref, w2_ref, b2_ref, ht_ref):
    h1 = jnp.dot(x_ref[...], w1_ref[...], preferred_element_type=jnp.float32)
    h1 = jnp.maximum(h1 + b1_ref[...], 0.0)
    h2 = jnp.dot(h1, w2_ref[...], preferred_element_type=jnp.float32)
    ht_ref[...] = jnp.maximum(h2 + b2_ref[...], 0.0).T


def _mlp_t(x, W1, b1, W2, b2):
    return pl.pallas_call(
        _mlp_body,
        out_shape=jax.ShapeDtypeStruct((D, N_NODES), jnp.float32),
    )(x, W1, b1.reshape(1, D), W2, b2.reshape(1, D))


def _edge_dot_body(ht_hbm, src_hbm, dst_hbm, pred_hbm,
                   acc_spmem, table, sidx, didx, partial, idrows, sem1, sem2):
    cid = lax.axis_index("c")
    sid = lax.axis_index("s")
    ebase_sc = cid * PER_SC
    rowbase_sc = cid * (PER_SC // 128)

    for j in range(WORDS):
        pltpu.sync_copy(ht_hbm.at[sid * WORDS + j],
                        table.at[pl.ds(j * N_NODES, N_NODES)])

    l16 = lax.iota(jnp.int32, 16)
    for rr in range(ROWS // 16):
        idrows[pl.ds(rr * 16, 16)] = l16 + (rr * 16)

    def chunk_body(c, carry):
        eoff = ebase_sc + c * CHUNK
        cp1 = pltpu.async_copy(src_hbm.at[pl.ds(eoff, CHUNK)], sidx, sem1)
        cp2 = pltpu.async_copy(dst_hbm.at[pl.ds(eoff, CHUNK)], didx, sem2)
        cp1.wait()
        cp2.wait()

        def row_body(r, rcarry):
            for gg in range(8):
                gb = r * 128 + gg * 16
                sids = sidx[pl.ds(gb, 16)]
                dids = didx[pl.ds(gb, 16)]
                acc = None
                for j in range(WORDS):
                    va = plsc.load_gather(table, [sids + (j * N_NODES)])
                    vb = plsc.load_gather(table, [dids + (j * N_NODES)])
                    term = va * vb
                    acc = term if acc is None else acc + term
                partial[r, pl.ds(gg * 16, 16)] = acc
            return rcarry

        lax.fori_loop(0, ROWS, row_body, 0)

        @pl.when(sid == 0)
        def _():
            pltpu.sync_copy(partial, acc_spmem)
        plsc.subcore_barrier()
        @pl.when(sid != 0)
        def _():
            pltpu.sync_copy(partial, acc_spmem.at[idrows], add=True)
        plsc.subcore_barrier()
        @pl.when(sid == 0)
        def _():
            pltpu.sync_copy(acc_spmem,
                            pred_hbm.at[pl.ds(rowbase_sc + c * ROWS, ROWS)])
        return carry

    lax.fori_loop(0, NCHUNK, chunk_body, 0)


def _edge_dots(ht, src, dst):
    mesh = plsc.VectorSubcoreMesh(core_axis_name="c", subcore_axis_name="s")
    k = functools.partial(
        pl.kernel,
        out_type=jax.ShapeDtypeStruct((E_PAD // 128, 128), jnp.float32),
        mesh=mesh,
        scratch_types=[
            pltpu.VMEM_SHARED((ROWS, 128), jnp.float32),
            pltpu.VMEM((WORDS * N_NODES,), jnp.float32),
            pltpu.VMEM((CHUNK,), jnp.int32),
            pltpu.VMEM((CHUNK,), jnp.int32),
            pltpu.VMEM((ROWS, 128), jnp.float32),
            pltpu.VMEM((ROWS,), jnp.int32),
            pltpu.SemaphoreType.DMA,
            pltpu.SemaphoreType.DMA,
        ],
        compiler_params=pltpu.CompilerParams(
            needs_layout_passes=False, use_tc_tiling_on_sc=False),
    )(_edge_dot_body)
    return k(ht, src, dst)


def kernel(x, edge_index, edge_label, W1, b1, W2, b2):
    ht = _mlp_t(x, W1, b1, W2, b2)
    ei = edge_index.astype(jnp.int32)
    pad = E_PAD - N_EDGES
    src = jnp.pad(ei[0], (0, pad))
    dst = jnp.pad(ei[1], (0, pad))
    pred2d = _edge_dots(ht, src, dst)
    return (pred2d.reshape(E_PAD)[:N_EDGES], edge_label)


# speedup vs baseline: 2.8492x; 1.2084x over previous
import functools

import jax
import jax.numpy as jnp
from jax import lax
from jax.experimental import pallas as pl
from jax.experimental.pallas import tpu as pltpu
from jax.experimental.pallas import tpu_sc as plsc

N_NODES = 10000
D = 128
N_EDGES = 320000

NSC = 2
NTILE = 16
WPT = D // (2 * NTILE) // 2

WORDS = D // NTILE

CHUNK = 4096
ROWS = CHUNK // 128
PER_SC = 163840
NCHUNK = PER_SC // CHUNK
E_PAD = PER_SC * NSC

ROW_BLOCK = 1000


def _mlp_body(x_ref, w1_ref, b1_ref, w2_ref, b2_ref, ht_ref):
    h1 = jnp.dot(x_ref[...], w1_ref[...], preferred_element_type=jnp.float32)
    h1 = jnp.maximum(h1 + b1_ref[...], 0.0)
    h2 = jnp.dot(h1, w2_ref[...], preferred_element_type=jnp.float32)
    ht_ref[...] = jnp.maximum(h2 + b2_ref[...], 0.0).T


def _mlp_t(x, W1, b1, W2, b2):
    return pl.pallas_call(
        _mlp_body,
        out_shape=jax.ShapeDtypeStruct((D, N_NODES), jnp.float32),
    )(x, W1, b1.reshape(1, D), W2, b2.reshape(1, D))


def _edge_dot_body(ht_hbm, src_hbm, dst_hbm, pred_hbm,
                   acc_spmem, table, sidx0, sidx1, didx0, didx1,
                   partial, zrows, idrows, sem_s0, sem_s1, sem_d0, sem_d1):
    cid = lax.axis_index("c")
    sid = lax.axis_index("s")
    ebase_sc = cid * PER_SC
    rowbase_sc = cid * (PER_SC // 128)
    sidx = (sidx0, sidx1)
    didx = (didx0, didx1)
    sem_s = (sem_s0, sem_s1)
    sem_d = (sem_d0, sem_d1)
    rpt = ROWS // NTILE

    for j in range(WORDS):
        pltpu.sync_copy(ht_hbm.at[sid * WORDS + j],
                        table.at[pl.ds(j * N_NODES, N_NODES)])

    l16 = lax.iota(jnp.int32, 16)
    for rr in range(ROWS // 16):
        idrows[pl.ds(rr * 16, 16)] = l16 + (rr * 16)
    for rr in range(rpt):
        for j in range(128 // 16):
            zrows[rr, pl.ds(j * 16, 16)] = jnp.zeros((16,), jnp.float32)
    pltpu.sync_copy(zrows, acc_spmem.at[pl.ds(sid * rpt, rpt)])

    pltpu.async_copy(src_hbm.at[pl.ds(ebase_sc, CHUNK)], sidx0, sem_s0)
    pltpu.async_copy(dst_hbm.at[pl.ds(ebase_sc, CHUNK)], didx0, sem_d0)

    def compute_partial(six, dix):
        def row_body(r, rcarry):
            for gg in range(8):
                gb = r * 128 + gg * 16
                sids = six[pl.ds(gb, 16)]
                dids = dix[pl.ds(gb, 16)]
                acc = None
                for j in range(WORDS):
                    va = plsc.load_gather(table, [sids + (j * N_NODES)])
                    vb = plsc.load_gather(table, [dids + (j * N_NODES)])
                    term = va * vb
                    acc = term if acc is None else acc + term
                partial[r, pl.ds(gg * 16, 16)] = acc
            return rcarry

        lax.fori_loop(0, ROWS, row_body, 0)

    def pair_body(c2, carry):
        for b in range(2):
            c = c2 * 2 + b
            nb = 1 - b

            @pl.when(c + 1 < NCHUNK)
            def _():
                eoff = ebase_sc + (c + 1) * CHUNK
                pltpu.async_copy(src_hbm.at[pl.ds(eoff, CHUNK)],
                                 sidx[nb], sem_s[nb])
                pltpu.async_copy(dst_hbm.at[pl.ds(eoff, CHUNK)],
                                 didx[nb], sem_d[nb])

            pltpu.make_async_copy(
                src_hbm.at[pl.ds(ebase_sc, CHUNK)], sidx[b], sem_s[b]).wait()
            pltpu.make_async_copy(
                dst_hbm.at[pl.ds(ebase_sc, CHUNK)], didx[b], sem_d[b]).wait()

            compute_partial(sidx[b], didx[b])

            plsc.subcore_barrier()
            pltpu.sync_copy(partial, acc_spmem.at[idrows], add=True)
            plsc.subcore_barrier()
            rbase = sid * rpt
            pltpu.sync_copy(
                acc_spmem.at[pl.ds(rbase, rpt)],
                pred_hbm.at[pl.ds(rowbase_sc + c * ROWS + rbase, rpt)])
            pltpu.sync_copy(zrows, acc_spmem.at[pl.ds(rbase, rpt)])
        return carry

    lax.fori_loop(0, NCHUNK // 2, pair_body, 0)


def _edge_dots(ht, src, dst):
    mesh = plsc.VectorSubcoreMesh(core_axis_name="c", subcore_axis_name="s")
    k = functools.partial(
        pl.kernel,
        out_type=jax.ShapeDtypeStruct((E_PAD // 128, 128), jnp.float32),
        mesh=mesh,
        scratch_types=[
            pltpu.VMEM_SHARED((ROWS, 128), jnp.float32),
            pltpu.VMEM((WORDS * N_NODES,), jnp.float32),
            pltpu.VMEM((CHUNK,), jnp.int32),
            pltpu.VMEM((CHUNK,), jnp.int32),
            pltpu.VMEM((CHUNK,), jnp.int32),
            pltpu.VMEM((CHUNK,), jnp.int32),
            pltpu.VMEM((ROWS, 128), jnp.float32),
            pltpu.VMEM((ROWS // NTILE, 128), jnp.float32),
            pltpu.VMEM((ROWS,), jnp.int32),
            pltpu.SemaphoreType.DMA,
            pltpu.SemaphoreType.DMA,
            pltpu.SemaphoreType.DMA,
            pltpu.SemaphoreType.DMA,
        ],
        compiler_params=pltpu.CompilerParams(
            needs_layout_passes=False, use_tc_tiling_on_sc=False),
    )(_edge_dot_body)
    return k(ht, src, dst)


def kernel(x, edge_index, edge_label, W1, b1, W2, b2):
    ht = _mlp_t(x, W1, b1, W2, b2)
    ei = edge_index.astype(jnp.int32)
    pad = E_PAD - N_EDGES
    src = jnp.pad(ei[0], (0, pad))
    dst = jnp.pad(ei[1], (0, pad))
    pred2d = _edge_dots(ht, src, dst)
    return (pred2d.reshape(E_PAD)[:N_EDGES], edge_label)


# speedup vs baseline: 3.0192x; 1.0597x over previous
import functools

import jax
import jax.numpy as jnp
from jax import lax
from jax.experimental import pallas as pl
from jax.experimental.pallas import tpu as pltpu
from jax.experimental.pallas import tpu_sc as plsc

N_NODES = 10000
D = 128
N_EDGES = 320000

NSC = 2
NTILE = 16
WPT = D // (2 * NTILE) // 2

WORDS = D // NTILE

CHUNK = 8192
ROWS = CHUNK // 128
PER_SC = 163840
NCHUNK = PER_SC // CHUNK
E_PAD = PER_SC * NSC

ROW_BLOCK = 1000


def _mlp_body(x_ref, w1_ref, b1_ref, w2_ref, b2_ref, ht_ref):
    h1 = jnp.dot(x_ref[...], w1_ref[...], preferred_element_type=jnp.float32)
    h1 = jnp.maximum(h1 + b1_ref[...], 0.0)
    h2 = jnp.dot(h1, w2_ref[...], preferred_element_type=jnp.float32)
    ht_ref[...] = jnp.maximum(h2 + b2_ref[...], 0.0).T


def _mlp_t(x, W1, b1, W2, b2):
    return pl.pallas_call(
        _mlp_body,
        out_shape=jax.ShapeDtypeStruct((D, N_NODES), jnp.float32),
    )(x, W1, b1.reshape(1, D), W2, b2.reshape(1, D))


def _edge_dot_body(ht_hbm, src_hbm, dst_hbm, pred_hbm,
                   acc_spmem, table, sidx0, sidx1, didx0, didx1,
                   partial, zrows, idrows, sem_s0, sem_s1, sem_d0, sem_d1):
    cid = lax.axis_index("c")
    sid = lax.axis_index("s")
    ebase_sc = cid * PER_SC
    rowbase_sc = cid * (PER_SC // 128)
    sidx = (sidx0, sidx1)
    didx = (didx0, didx1)
    sem_s = (sem_s0, sem_s1)
    sem_d = (sem_d0, sem_d1)
    rpt = ROWS // NTILE

    for j in range(WORDS):
        pltpu.sync_copy(ht_hbm.at[sid * WORDS + j],
                        table.at[pl.ds(j * N_NODES, N_NODES)])

    l16 = lax.iota(jnp.int32, 16)
    for rr in range(ROWS // 16):
        idrows[pl.ds(rr * 16, 16)] = l16 + (rr * 16)
    for rr in range(rpt):
        for j in range(128 // 16):
            zrows[rr, pl.ds(j * 16, 16)] = jnp.zeros((16,), jnp.float32)
    pltpu.sync_copy(zrows, acc_spmem.at[pl.ds(sid * rpt, rpt)])

    pltpu.async_copy(src_hbm.at[pl.ds(ebase_sc, CHUNK)], sidx0, sem_s0)
    pltpu.async_copy(dst_hbm.at[pl.ds(ebase_sc, CHUNK)], didx0, sem_d0)

    def compute_partial(six, dix):
        def row_body(r, rcarry):
            for gg in range(8):
                gb = r * 128 + gg * 16
                sids = six[pl.ds(gb, 16)]
                dids = dix[pl.ds(gb, 16)]
                acc = None
                for j in range(WORDS):
                    va = plsc.load_gather(table, [sids + (j * N_NODES)])
                    vb = plsc.load_gather(table, [dids + (j * N_NODES)])
                    term = va * vb
                    acc = term if acc is None else acc + term
                partial[r, pl.ds(gg * 16, 16)] = acc
            return rcarry

        lax.fori_loop(0, ROWS, row_body, 0)

    def pair_body(c2, carry):
        for b in range(2):
            c = c2 * 2 + b
            nb = 1 - b

            @pl.when(c + 1 < NCHUNK)
            def _():
                eoff = ebase_sc + (c + 1) * CHUNK
                pltpu.async_copy(src_hbm.at[pl.ds(eoff, CHUNK)],
                                 sidx[nb], sem_s[nb])
                pltpu.async_copy(dst_hbm.at[pl.ds(eoff, CHUNK)],
                                 didx[nb], sem_d[nb])

            pltpu.make_async_copy(
                src_hbm.at[pl.ds(ebase_sc, CHUNK)], sidx[b], sem_s[b]).wait()
            pltpu.make_async_copy(
                dst_hbm.at[pl.ds(ebase_sc, CHUNK)], didx[b], sem_d[b]).wait()

            compute_partial(sidx[b], didx[b])

            plsc.subcore_barrier()
            pltpu.sync_copy(partial, acc_spmem.at[idrows], add=True)
            plsc.subcore_barrier()
            rbase = sid * rpt
            pltpu.sync_copy(
                acc_spmem.at[pl.ds(rbase, rpt)],
                pred_hbm.at[pl.ds(rowbase_sc + c * ROWS + rbase, rpt)])
            pltpu.sync_copy(zrows, acc_spmem.at[pl.ds(rbase, rpt)])
        return carry

    lax.fori_loop(0, NCHUNK // 2, pair_body, 0)


def _edge_dots(ht, src, dst):
    mesh = plsc.VectorSubcoreMesh(core_axis_name="c", subcore_axis_name="s")
    k = functools.partial(
        pl.kernel,
        out_type=jax.ShapeDtypeStruct((E_PAD // 128, 128), jnp.float32),
        mesh=mesh,
        scratch_types=[
            pltpu.VMEM_SHARED((ROWS, 128), jnp.float32),
            pltpu.VMEM((WORDS * N_NODES,), jnp.float32),
            pltpu.VMEM((CHUNK,), jnp.int32),
            pltpu.VMEM((CHUNK,), jnp.int32),
            pltpu.VMEM((CHUNK,), jnp.int32),
            pltpu.VMEM((CHUNK,), jnp.int32),
            pltpu.VMEM((ROWS, 128), jnp.float32),
            pltpu.VMEM((ROWS // NTILE, 128), jnp.float32),
            pltpu.VMEM((ROWS,), jnp.int32),
            pltpu.SemaphoreType.DMA,
            pltpu.SemaphoreType.DMA,
            pltpu.SemaphoreType.DMA,
            pltpu.SemaphoreType.DMA,
        ],
        compiler_params=pltpu.CompilerParams(
            needs_layout_passes=False, use_tc_tiling_on_sc=False),
    )(_edge_dot_body)
    return k(ht, src, dst)


def kernel(x, edge_index, edge_label, W1, b1, W2, b2):
    ht = _mlp_t(x, W1, b1, W2, b2)
    ei = edge_index.astype(jnp.int32)
    pad = E_PAD - N_EDGES
    src = jnp.pad(ei[0], (0, pad))
    dst = jnp.pad(ei[1], (0, pad))
    pred2d = _edge_dots(ht, src, dst)
    return (pred2d.reshape(E_PAD)[:N_EDGES], edge_label)


# speedup vs baseline: 3.1446x; 1.0415x over previous
import functools

import jax
import jax.numpy as jnp
from jax import lax
from jax.experimental import pallas as pl
from jax.experimental.pallas import tpu as pltpu
from jax.experimental.pallas import tpu_sc as plsc

N_NODES = 10000
D = 128
N_EDGES = 320000

NSC = 2
NTILE = 16
WPT = D // (2 * NTILE) // 2

WORDS = D // NTILE // 2

CHUNK = 8192
ROWS = CHUNK // 128
PER_SC = 163840
NCHUNK = PER_SC // CHUNK
E_PAD = PER_SC * NSC

ROW_BLOCK = 1000


def _mlp_body(x_ref, w1_ref, b1_ref, w2_ref, b2_ref, ht_ref):
    h1 = jnp.dot(x_ref[...], w1_ref[...], preferred_element_type=jnp.float32)
    h1 = jnp.maximum(h1 + b1_ref[...], 0.0)
    h2 = jnp.dot(h1, w2_ref[...], preferred_element_type=jnp.float32)
    ht_ref[...] = jnp.maximum(h2 + b2_ref[...], 0.0).T.astype(jnp.bfloat16)


def _mlp_t(x, W1, b1, W2, b2):
    return pl.pallas_call(
        _mlp_body,
        out_shape=jax.ShapeDtypeStruct((D, N_NODES), jnp.bfloat16),
    )(x, W1, b1.reshape(1, D), W2, b2.reshape(1, D))


def _edge_dot_body(ht_hbm, src_hbm, dst_hbm, pred_hbm,
                   acc_spmem, table, sidx0, sidx1, didx0, didx1,
                   partial, zrows, idrows, sem_s0, sem_s1, sem_d0, sem_d1):
    cid = lax.axis_index("c")
    sid = lax.axis_index("s")
    ebase_sc = cid * PER_SC
    rowbase_sc = cid * (PER_SC // 128)
    sidx = (sidx0, sidx1)
    didx = (didx0, didx1)
    sem_s = (sem_s0, sem_s1)
    sem_d = (sem_d0, sem_d1)
    rpt = ROWS // NTILE

    for j in range(WORDS):
        pltpu.sync_copy(ht_hbm.at[sid * WORDS + j],
                        table.at[pl.ds(j * N_NODES, N_NODES)])

    l16 = lax.iota(jnp.int32, 16)
    for rr in range(ROWS // 16):
        idrows[pl.ds(rr * 16, 16)] = l16 + (rr * 16)
    for rr in range(rpt):
        for j in range(128 // 16):
            zrows[rr, pl.ds(j * 16, 16)] = jnp.zeros((16,), jnp.float32)
    pltpu.sync_copy(zrows, acc_spmem.at[pl.ds(sid * rpt, rpt)])

    pltpu.async_copy(src_hbm.at[pl.ds(ebase_sc, CHUNK)], sidx0, sem_s0)
    pltpu.async_copy(dst_hbm.at[pl.ds(ebase_sc, CHUNK)], didx0, sem_d0)

    def compute_partial(six, dix):
        def row_body(r, rcarry):
            for gg in range(8):
                gb = r * 128 + gg * 16
                sids = six[pl.ds(gb, 16)]
                dids = dix[pl.ds(gb, 16)]
                acc = None
                for j in range(WORDS):
                    va = plsc.load_gather(table, [sids + (j * N_NODES)])
                    vb = plsc.load_gather(table, [dids + (j * N_NODES)])
                    a0, a1 = plsc.unpack(plsc.bitcast(va, jnp.bfloat16),
                                         format=plsc.PackFormat.INTERLEAVED)
                    b0, b1 = plsc.unpack(plsc.bitcast(vb, jnp.bfloat16),
                                         format=plsc.PackFormat.INTERLEAVED)
                    term = a0 * b0 + a1 * b1
                    acc = term if acc is None else acc + term
                partial[r, pl.ds(gg * 16, 16)] = acc
            return rcarry

        lax.fori_loop(0, ROWS, row_body, 0)

    def pair_body(c2, carry):
        for b in range(2):
            c = c2 * 2 + b
            nb = 1 - b

            @pl.when(c + 1 < NCHUNK)
            def _():
                eoff = ebase_sc + (c + 1) * CHUNK
                pltpu.async_copy(src_hbm.at[pl.ds(eoff, CHUNK)],
                                 sidx[nb], sem_s[nb])
                pltpu.async_copy(dst_hbm.at[pl.ds(eoff, CHUNK)],
                                 didx[nb], sem_d[nb])

            pltpu.make_async_copy(
                src_hbm.at[pl.ds(ebase_sc, CHUNK)], sidx[b], sem_s[b]).wait()
            pltpu.make_async_copy(
                dst_hbm.at[pl.ds(ebase_sc, CHUNK)], didx[b], sem_d[b]).wait()

            compute_partial(sidx[b], didx[b])

            plsc.subcore_barrier()
            pltpu.sync_copy(partial, acc_spmem.at[idrows], add=True)
            plsc.subcore_barrier()
            rbase = sid * rpt
            pltpu.sync_copy(
                acc_spmem.at[pl.ds(rbase, rpt)],
                pred_hbm.at[pl.ds(rowbase_sc + c * ROWS + rbase, rpt)])
            pltpu.sync_copy(zrows, acc_spmem.at[pl.ds(rbase, rpt)])
        return carry

    lax.fori_loop(0, NCHUNK // 2, pair_body, 0)


def _edge_dots(ht, src, dst):
    mesh = plsc.VectorSubcoreMesh(core_axis_name="c", subcore_axis_name="s")
    k = functools.partial(
        pl.kernel,
        out_type=jax.ShapeDtypeStruct((E_PAD // 128, 128), jnp.float32),
        mesh=mesh,
        scratch_types=[
            pltpu.VMEM_SHARED((ROWS, 128), jnp.float32),
            pltpu.VMEM((WORDS * N_NODES,), jnp.int32),
            pltpu.VMEM((CHUNK,), jnp.int32),
            pltpu.VMEM((CHUNK,), jnp.int32),
            pltpu.VMEM((CHUNK,), jnp.int32),
            pltpu.VMEM((CHUNK,), jnp.int32),
            pltpu.VMEM((ROWS, 128), jnp.float32),
            pltpu.VMEM((ROWS // NTILE, 128), jnp.float32),
            pltpu.VMEM((ROWS,), jnp.int32),
            pltpu.SemaphoreType.DMA,
            pltpu.SemaphoreType.DMA,
            pltpu.SemaphoreType.DMA,
            pltpu.SemaphoreType.DMA,
        ],
        compiler_params=pltpu.CompilerParams(
            needs_layout_passes=False, use_tc_tiling_on_sc=False),
    )(_edge_dot_body)
    return k(ht, src, dst)


def kernel(x, edge_index, edge_label, W1, b1, W2, b2):
    ht_bf = _mlp_t(x, W1, b1, W2, b2)
    ht = lax.bitcast_convert_type(
        ht_bf.reshape(D // 2, 2, N_NODES).transpose(0, 2, 1), jnp.int32)
    ei = edge_index.astype(jnp.int32)
    pad = E_PAD - N_EDGES
    src = jnp.pad(ei[0], (0, pad))
    dst = jnp.pad(ei[1], (0, pad))
    pred2d = _edge_dots(ht, src, dst)
    return (pred2d.reshape(E_PAD)[:N_EDGES], edge_label)
